# baseline (speedup 1.0000x reference)
import jax
import jax.numpy as jnp
from jax import lax
from jax.experimental import pallas as pl
from jax.experimental.pallas import tpu as pltpu
from jax.experimental.pallas import tpu_sc as plsc

VOCAB = 1000000
HIDDEN = 64
NUM_CHANNEL = 4
STEPS = 200
BATCH = 1024

NW = 32
BATCH_PER_W = BATCH // NW
SUB = 40
NSUB = STEPS // SUB
PAIR = 2 * HIDDEN
L = 16
HV = HIDDEN // L


def _rsqrt(v16):
    bits = lax.bitcast_convert_type(v16, jnp.int32)
    y = lax.bitcast_convert_type(
        jnp.int32(0x5F3759DF) - lax.shift_right_logical(bits, 1), jnp.float32)
    for _ in range(3):
        y = y * (1.5 - 0.5 * v16 * y * y)
    return y


def _body(x_hbm, chw_hbm, gamma_hbm, beta_hbm, pos_hbm, table_hbm,
          out_hbm, idx_v, idxh_v, rows_v, out_v, pos_v, gb_v, w_v,
          s0, s1, s2, s3, s4):
    wid = lax.axis_index("s") * 2 + lax.axis_index("c")
    sems = [s0, s1, s2, s3, s4]

    pltpu.sync_copy(pos_hbm, pos_v)
    pltpu.sync_copy(gamma_hbm, gb_v.at[0])
    pltpu.sync_copy(beta_hbm, gb_v.at[1])
    pltpu.sync_copy(chw_hbm, w_v)

    e = jnp.exp(w_v[...])
    w = e / jnp.sum(e)
    ws = [w[c] for c in range(NUM_CHANNEL)]
    gam = [gb_v[0, pl.ds(k * L, L)] for k in range(HV)]
    bet = [gb_v[1, pl.ds(k * L, L)] for k in range(HV)]

    @pl.loop(0, BATCH_PER_W)
    def batch_loop(bl):
        b = wid * BATCH_PER_W + bl
        for c in range(NUM_CHANNEL):
            pltpu.sync_copy(x_hbm.at[b, c], idx_v.at[c, pl.ds(0, STEPS)])
        for c in range(NUM_CHANNEL):
            for t in range(13):
                st = 16 * t if t < 12 else STEPS - 8
                v = idx_v[c, pl.ds(st, L)]
                idxh_v[c, pl.ds(st, L)] = lax.shift_right_logical(v, 1)
        cps = []
        for si in range(NSUB):
            for c in range(NUM_CHANNEL):
                cps.append(pltpu.async_copy(
                    table_hbm.at[idxh_v.at[c, pl.ds(si * SUB, SUB)]],
                    rows_v.at[c * NSUB + si], sems[si]))

        for si in range(NSUB):
            for c in range(NUM_CHANNEL):
                cps[si * NUM_CHANNEL + c].wait()

            @pl.loop(0, SUB, unroll=4)
            def pos_loop(j, _si=si):
                s = _si * SUB + j
                po = [lax.shift_left(idx_v[c, pl.ds(s, L)][0] & 1, 6)
                      for c in range(NUM_CHANNEL)]
                acc = [ws[0] * rows_v[_si, j, pl.ds(po[0] + k * L, L)]
                       + ws[1] * rows_v[NSUB + _si, j, pl.ds(po[1] + k * L, L)]
                       + ws[2] * rows_v[2 * NSUB + _si, j, pl.ds(po[2] + k * L, L)]
                       + ws[3] * rows_v[3 * NSUB + _si, j, pl.ds(po[3] + k * L, L)]
                       for k in range(HV)]
                tot = (acc[0] + acc[1]) + (acc[2] + acc[3])
                sq = (acc[0] * acc[0] + acc[1] * acc[1]) + \
                     (acc[2] * acc[2] + acc[3] * acc[3])
                mean = jnp.sum(tot) * (1.0 / HIDDEN)
                var = jnp.sum(sq) * (1.0 / HIDDEN) - mean * mean
                rstd = _rsqrt(jnp.full((L,), var + 1e-5, jnp.float32))
                for k in range(HV):
                    out_v[s, pl.ds(k * L, L)] = (
                        (acc[k] - mean) * rstd * gam[k] + bet[k]
                        + pos_v[s, pl.ds(k * L, L)])

        pltpu.sync_copy(out_v, out_hbm.at[b])


@jax.jit
def kernel(x, table, ch_w, ln_gamma, ln_beta, pos_emb):
    table2 = table.reshape(VOCAB // 2, PAIR)
    chw16 = jnp.full((L,), -1e30, jnp.float32).at[:NUM_CHANNEL].set(ch_w)

    mesh = plsc.VectorSubcoreMesh(core_axis_name="c", subcore_axis_name="s")
    run = pl.kernel(
        _body,
        out_type=jax.ShapeDtypeStruct((BATCH, STEPS, HIDDEN), jnp.float32),
        mesh=mesh,
        scratch_types=[
            pltpu.VMEM((NUM_CHANNEL, 224), jnp.int32),
            pltpu.VMEM((NUM_CHANNEL, 208), jnp.int32),
            pltpu.VMEM((NUM_CHANNEL * NSUB, SUB, PAIR), jnp.float32),
            pltpu.VMEM((STEPS, HIDDEN), jnp.float32),
            pltpu.VMEM((STEPS, HIDDEN), jnp.float32),
            pltpu.VMEM((2, HIDDEN), jnp.float32),
            pltpu.VMEM((L,), jnp.float32),
            pltpu.SemaphoreType.DMA,
            pltpu.SemaphoreType.DMA,
            pltpu.SemaphoreType.DMA,
            pltpu.SemaphoreType.DMA,
            pltpu.SemaphoreType.DMA,
        ],
        compiler_params=pltpu.CompilerParams(
            needs_layout_passes=False, use_tc_tiling_on_sc=False),
    )
    return run(x, chw16, ln_gamma, ln_beta, pos_emb, table2)


# speedup vs baseline: 1.0042x; 1.0042x over previous
import jax
import jax.numpy as jnp
import numpy as np
from jax import lax
from jax.experimental import pallas as pl
from jax.experimental.pallas import tpu as pltpu
from jax.experimental.pallas import tpu_sc as plsc

VOCAB = 1000000
HIDDEN = 64
NUM_CHANNEL = 4
STEPS = 200
BATCH = 1024

NW = 32
BATCH_PER_W = BATCH // NW
SUB = 40
NSUB = STEPS // SUB
L = 16
HH = HIDDEN // 2

PERM = np.concatenate([np.arange(0, 32, 2), np.arange(1, 32, 2),
                       np.arange(32, 64, 2), np.arange(33, 64, 2)])


def _rsqrt(v16):
    bits = lax.bitcast_convert_type(v16, jnp.int32)
    y = lax.bitcast_convert_type(
        jnp.int32(0x5F3759DF) - lax.shift_right_logical(bits, 1), jnp.float32)
    for _ in range(3):
        y = y * (1.5 - 0.5 * v16 * y * y)
    return y


def _body(x_hbm, chw_hbm, gamma_hbm, beta_hbm, pos_hbm, table_hbm,
          out_hbm, idx_v, rows_v, out_v, pos_v, gb_v, w_v, sem):
    wid = lax.axis_index("s") * 2 + lax.axis_index("c")

    pltpu.sync_copy(pos_hbm, pos_v)
    pltpu.sync_copy(gamma_hbm, gb_v.at[0])
    pltpu.sync_copy(beta_hbm, gb_v.at[1])
    pltpu.sync_copy(chw_hbm, w_v)

    e = jnp.exp(w_v[...])
    w = e / jnp.sum(e)
    ws = [w[c] for c in range(NUM_CHANNEL)]
    gam = [gb_v[0, pl.ds(k * L, L)] for k in range(4)]
    bet = [gb_v[1, pl.ds(k * L, L)] for k in range(4)]
    iota = lax.iota(jnp.int32, L)
    hvec = [2 * iota, 2 * iota + 1, 2 * iota + 32, 2 * iota + 33]
    mhi = jnp.full((L,), jnp.int32(-65536))

    @pl.loop(0, BATCH_PER_W)
    def batch_loop(bl):
        b = wid * BATCH_PER_W + bl
        pltpu.sync_copy(x_hbm.at[b], idx_v)
        cps = [pltpu.async_copy(
                   table_hbm.at[idx_v.at[c, pl.ds(si * SUB, SUB)]],
                   rows_v.at[c * NSUB + si], sem)
               for c in range(NUM_CHANNEL) for si in range(NSUB)]
        for cp in cps:
            cp.wait()

        for si in range(NSUB):
            @pl.loop(0, SUB, unroll=4)
            def pos_loop(j, _si=si):
                acc = [None] * 4
                for c in range(NUM_CHANNEL):
                    for m in range(2):
                        raw = plsc.bitcast(
                            rows_v[c * NSUB + _si, j, pl.ds(m * HH, HH)],
                            jnp.int32)
                        ev = lax.bitcast_convert_type(
                            lax.shift_left(raw, 16), jnp.float32)
                        od = lax.bitcast_convert_type(raw & mhi, jnp.float32)
                        if c == 0:
                            acc[2 * m] = ws[0] * ev
                            acc[2 * m + 1] = ws[0] * od
                        else:
                            acc[2 * m] += ws[c] * ev
                            acc[2 * m + 1] += ws[c] * od
                tot = (acc[0] + acc[1]) + (acc[2] + acc[3])
                sq = (acc[0] * acc[0] + acc[1] * acc[1]) + \
                     (acc[2] * acc[2] + acc[3] * acc[3])
                mean = jnp.sum(tot) * (1.0 / HIDDEN)
                var = jnp.sum(sq) * (1.0 / HIDDEN) - mean * mean
                rstd = _rsqrt(jnp.full((L,), var + 1e-5, jnp.float32))
                s = _si * SUB + j
                srow = jnp.full((L,), s, jnp.int32)
                for k in range(4):
                    val = ((acc[k] - mean) * rstd * gam[k] + bet[k]
                           + pos_v[s, pl.ds(k * L, L)])
                    plsc.store_scatter(out_v, [srow, hvec[k]], val)

        pltpu.sync_copy(out_v, out_hbm.at[b])


@jax.jit
def kernel(x, table, ch_w, ln_gamma, ln_beta, pos_emb):
    table_bf = table.astype(jnp.bfloat16)
    chw16 = jnp.full((L,), -1e30, jnp.float32).at[:NUM_CHANNEL].set(ch_w)
    perm = jnp.asarray(PERM)
    gam_sh = ln_gamma[perm]
    bet_sh = ln_beta[perm]
    pos_sh = pos_emb[:, perm]

    mesh = plsc.VectorSubcoreMesh(core_axis_name="c", subcore_axis_name="s")
    run = pl.kernel(
        _body,
        out_type=jax.ShapeDtypeStruct((BATCH, STEPS, HIDDEN), jnp.float32),
        mesh=mesh,
        scratch_types=[
            pltpu.VMEM((NUM_CHANNEL, STEPS), jnp.int32),
            pltpu.VMEM((NUM_CHANNEL * NSUB, SUB, HIDDEN), jnp.bfloat16),
            pltpu.VMEM((STEPS, HIDDEN), jnp.float32),
            pltpu.VMEM((STEPS, HIDDEN), jnp.float32),
            pltpu.VMEM((2, HIDDEN), jnp.float32),
            pltpu.VMEM((L,), jnp.float32),
            pltpu.SemaphoreType.DMA,
        ],
        compiler_params=pltpu.CompilerParams(
            needs_layout_passes=False, use_tc_tiling_on_sc=False),
    )
    return run(x, chw16, gam_sh, bet_sh, pos_sh, table_bf)


# speedup vs baseline: 1.4737x; 1.4676x over previous
import jax
import jax.numpy as jnp
from jax import lax
from jax.experimental import pallas as pl
from jax.experimental.pallas import tpu as pltpu
from jax.experimental.pallas import tpu_sc as plsc

VOCAB = 1000000
HIDDEN = 64
NUM_CHANNEL = 4
STEPS = 200
BATCH = 1024

NW = 32
BATCH_PER_W = BATCH // NW
SUB = 40
NSUB = STEPS // SUB
L = 16
HV = HIDDEN // L


def _rsqrt(v16):
    bits = lax.bitcast_convert_type(v16, jnp.int32)
    y = lax.bitcast_convert_type(
        jnp.int32(0x5F3759DF) - lax.shift_right_logical(bits, 1), jnp.float32)
    for _ in range(3):
        y = y * (1.5 - 0.5 * v16 * y * y)
    return y


def _body(x_hbm, chw_hbm, gamma_hbm, beta_hbm, pos_hbm, table_hbm,
          out_hbm, idx0, idx1, rows0, rows1, out_v, pos_v, gb_v, w_v,
          sem0, sem1):
    wid = lax.axis_index("s") * 2 + lax.axis_index("c")

    pltpu.sync_copy(pos_hbm, pos_v)
    pltpu.sync_copy(gamma_hbm, gb_v.at[0])
    pltpu.sync_copy(beta_hbm, gb_v.at[1])
    pltpu.sync_copy(chw_hbm, w_v)

    e = jnp.exp(w_v[...])
    w = e / jnp.sum(e)
    ws = [w[c] for c in range(NUM_CHANNEL)]
    gam = [gb_v[0, pl.ds(k * L, L)] for k in range(HV)]
    bet = [gb_v[1, pl.ds(k * L, L)] for k in range(HV)]

    def fire(b, idx_v, rows_v, sem):
        pltpu.sync_copy(x_hbm.at[b], idx_v)
        return [pltpu.async_copy(
                    table_hbm.at[idx_v.at[c, pl.ds(si * SUB, SUB)]],
                    rows_v.at[c * NSUB + si], sem)
                for c in range(NUM_CHANNEL) for si in range(NSUB)]

    def compute(rows_v, b):
        for si in range(NSUB):
            @pl.loop(0, SUB, unroll=4)
            def pos_loop(j, _si=si, _rows=rows_v):
                acc = [ws[0] * _rows[_si, j, pl.ds(k * L, L)]
                       + ws[1] * _rows[NSUB + _si, j, pl.ds(k * L, L)]
                       + ws[2] * _rows[2 * NSUB + _si, j, pl.ds(k * L, L)]
                       + ws[3] * _rows[3 * NSUB + _si, j, pl.ds(k * L, L)]
                       for k in range(HV)]
                tot = (acc[0] + acc[1]) + (acc[2] + acc[3])
                sq = (acc[0] * acc[0] + acc[1] * acc[1]) + \
                     (acc[2] * acc[2] + acc[3] * acc[3])
                mean = jnp.sum(tot) * (1.0 / HIDDEN)
                var = jnp.sum(sq) * (1.0 / HIDDEN) - mean * mean
                rstd = _rsqrt(jnp.full((L,), var + 1e-5, jnp.float32))
                s = _si * SUB + j
                for k in range(HV):
                    out_v[j, pl.ds(k * L, L)] = (
                        (acc[k] - mean) * rstd * gam[k] + bet[k]
                        + pos_v[s, pl.ds(k * L, L)])

            pltpu.sync_copy(out_v, out_hbm.at[b, pl.ds(si * SUB, SUB)])

    b_base = wid * BATCH_PER_W

    @pl.loop(0, BATCH_PER_W // 2)
    def pair_loop(p):
        b0 = b_base + 2 * p
        cps0 = fire(b0, idx0, rows0, sem0)
        cps1 = fire(b0 + 1, idx1, rows1, sem1)
        for cp in cps0:
            cp.wait()
        compute(rows0, b0)
        for cp in cps1:
            cp.wait()
        compute(rows1, b0 + 1)


@jax.jit
def kernel(x, table, ch_w, ln_gamma, ln_beta, pos_emb):
    chw16 = jnp.full((L,), -1e30, jnp.float32).at[:NUM_CHANNEL].set(ch_w)

    mesh = plsc.VectorSubcoreMesh(core_axis_name="c", subcore_axis_name="s")
    run = pl.kernel(
        _body,
        out_type=jax.ShapeDtypeStruct((BATCH, STEPS, HIDDEN), jnp.float32),
        mesh=mesh,
        scratch_types=[
            pltpu.VMEM((NUM_CHANNEL, STEPS), jnp.int32),
            pltpu.VMEM((NUM_CHANNEL, STEPS), jnp.int32),
            pltpu.VMEM((NUM_CHANNEL * NSUB, SUB, HIDDEN), jnp.float32),
            pltpu.VMEM((NUM_CHANNEL * NSUB, SUB, HIDDEN), jnp.float32),
            pltpu.VMEM((SUB, HIDDEN), jnp.float32),
            pltpu.VMEM((STEPS, HIDDEN), jnp.float32),
            pltpu.VMEM((2, HIDDEN), jnp.float32),
            pltpu.VMEM((L,), jnp.float32),
            pltpu.SemaphoreType.DMA,
            pltpu.SemaphoreType.DMA,
        ],
        compiler_params=pltpu.CompilerParams(
            needs_layout_passes=False, use_tc_tiling_on_sc=False),
    )
    return run(x, chw16, ln_gamma, ln_beta, pos_emb, table)


# speedup vs baseline: 1.5180x; 1.0301x over previous
import jax
import jax.numpy as jnp
from jax import lax
from jax.experimental import pallas as pl
from jax.experimental.pallas import tpu as pltpu
from jax.experimental.pallas import tpu_sc as plsc

VOCAB = 1000000
HIDDEN = 64
NUM_CHANNEL = 4
STEPS = 200
BATCH = 1024

NW = 32
BATCH_PER_W = BATCH // NW
SUB = 40
NSUB = STEPS // SUB
L = 16
HV = HIDDEN // L


def _rsqrt(v16):
    bits = lax.bitcast_convert_type(v16, jnp.int32)
    y = lax.bitcast_convert_type(
        jnp.int32(0x5F3759DF) - lax.shift_right_logical(bits, 1), jnp.float32)
    for _ in range(3):
        y = y * (1.5 - 0.5 * v16 * y * y)
    return y


def _body(x_hbm, chw_hbm, gamma_hbm, beta_hbm, pos_hbm, table_hbm,
          out_hbm, idx0, idx1, rows0, rows1, out_v, pos_v, gb_v, w_v,
          sem0, sem1):
    wid = lax.axis_index("s") * 2 + lax.axis_index("c")

    pltpu.sync_copy(pos_hbm, pos_v)
    pltpu.sync_copy(gamma_hbm, gb_v.at[0])
    pltpu.sync_copy(beta_hbm, gb_v.at[1])
    pltpu.sync_copy(chw_hbm, w_v)

    e = jnp.exp(w_v[...])
    w = e / jnp.sum(e)
    ws = [w[c] for c in range(NUM_CHANNEL)]
    gam = [gb_v[0, pl.ds(k * L, L)] for k in range(HV)]
    bet = [gb_v[1, pl.ds(k * L, L)] for k in range(HV)]

    def fire(b, idx_v, rows_v, sem):
        pltpu.sync_copy(x_hbm.at[b], idx_v)
        return [pltpu.async_copy(
                    table_hbm.at[idx_v.at[c, pl.ds(si * SUB, SUB)]],
                    rows_v.at[c * NSUB + si], sem)
                for c in range(NUM_CHANNEL) for si in range(NSUB)]

    def compute(rows_v, b):
        for si in range(NSUB):
            @pl.loop(0, SUB, unroll=4)
            def pos_loop(j, _si=si, _rows=rows_v):
                acc = [ws[0] * _rows[_si, j, pl.ds(k * L, L)]
                       + ws[1] * _rows[NSUB + _si, j, pl.ds(k * L, L)]
                       + ws[2] * _rows[2 * NSUB + _si, j, pl.ds(k * L, L)]
                       + ws[3] * _rows[3 * NSUB + _si, j, pl.ds(k * L, L)]
                       for k in range(HV)]
                tot = (acc[0] + acc[1]) + (acc[2] + acc[3])
                sq = (acc[0] * acc[0] + acc[1] * acc[1]) + \
                     (acc[2] * acc[2] + acc[3] * acc[3])
                mean = jnp.sum(tot) * (1.0 / HIDDEN)
                var = jnp.sum(sq) * (1.0 / HIDDEN) - mean * mean
                rstd = _rsqrt(jnp.full((L,), var + 1e-5, jnp.float32))
                s = _si * SUB + j
                for k in range(HV):
                    out_v[pl.ds(j * HIDDEN + k * L, L)] = (
                        (acc[k] - mean) * rstd * gam[k] + bet[k]
                        + pos_v[s, pl.ds(k * L, L)])

            pltpu.sync_copy(
                out_v, out_hbm.at[b, pl.ds(si * SUB * HIDDEN, SUB * HIDDEN)])

    b_base = wid * BATCH_PER_W

    @pl.loop(0, BATCH_PER_W // 2)
    def pair_loop(p):
        b0 = b_base + 2 * p
        cps0 = fire(b0, idx0, rows0, sem0)
        cps1 = fire(b0 + 1, idx1, rows1, sem1)
        for cp in cps0:
            cp.wait()
        compute(rows0, b0)
        for cp in cps1:
            cp.wait()
        compute(rows1, b0 + 1)


@jax.jit
def kernel(x, table, ch_w, ln_gamma, ln_beta, pos_emb):
    chw16 = jnp.full((L,), -1e30, jnp.float32).at[:NUM_CHANNEL].set(ch_w)

    mesh = plsc.VectorSubcoreMesh(core_axis_name="c", subcore_axis_name="s")
    run = pl.kernel(
        _body,
        out_type=jax.ShapeDtypeStruct((BATCH, STEPS * HIDDEN), jnp.float32),
        mesh=mesh,
        scratch_types=[
            pltpu.VMEM((NUM_CHANNEL, STEPS), jnp.int32),
            pltpu.VMEM((NUM_CHANNEL, STEPS), jnp.int32),
            pltpu.VMEM((NUM_CHANNEL * NSUB, SUB, HIDDEN), jnp.float32),
            pltpu.VMEM((NUM_CHANNEL * NSUB, SUB, HIDDEN), jnp.float32),
            pltpu.VMEM((SUB * HIDDEN,), jnp.float32),
            pltpu.VMEM((STEPS, HIDDEN), jnp.float32),
            pltpu.VMEM((2, HIDDEN), jnp.float32),
            pltpu.VMEM((L,), jnp.float32),
            pltpu.SemaphoreType.DMA,
            pltpu.SemaphoreType.DMA,
        ],
        compiler_params=pltpu.CompilerParams(
            needs_layout_passes=False, use_tc_tiling_on_sc=False),
    )
    out = run(x, chw16, ln_gamma, ln_beta, pos_emb, table)
    return out.reshape(BATCH, STEPS, HIDDEN)


# speedup vs baseline: 1.5434x; 1.0168x over previous
import jax
import jax.numpy as jnp
from jax import lax
from jax.experimental import pallas as pl
from jax.experimental.pallas import tpu as pltpu
from jax.experimental.pallas import tpu_sc as plsc

VOCAB = 1000000
HIDDEN = 64
NUM_CHANNEL = 4
STEPS = 200
BATCH = 1024

NW = 32
BATCH_PER_W = BATCH // NW
SUB = 40
NSUB = STEPS // SUB
L = 16
HV = HIDDEN // L


def _rsqrt(v16):
    bits = lax.bitcast_convert_type(v16, jnp.int32)
    y = lax.bitcast_convert_type(
        jnp.int32(0x5F3759DF) - lax.shift_right_logical(bits, 1), jnp.float32)
    for _ in range(2):
        y = y * (1.5 - 0.5 * v16 * y * y)
    return y


def _body(x_hbm, chw_hbm, gamma_hbm, beta_hbm, pos_hbm, table_hbm,
          out_hbm, idx0, idx1, rows0, rows1, out_v, pos_v, gb_v, w_v,
          sem0, sem1):
    wid = lax.axis_index("s") * 2 + lax.axis_index("c")

    pltpu.sync_copy(pos_hbm, pos_v)
    pltpu.sync_copy(gamma_hbm, gb_v.at[0])
    pltpu.sync_copy(beta_hbm, gb_v.at[1])
    pltpu.sync_copy(chw_hbm, w_v)

    e = jnp.exp(w_v[...])
    w = e / jnp.sum(e)
    ws = [w[c] for c in range(NUM_CHANNEL)]
    gam = [gb_v[0, pl.ds(k * L, L)] for k in range(HV)]
    bet = [gb_v[1, pl.ds(k * L, L)] for k in range(HV)]

    def fire(b, idx_v, rows_v, sem):
        pltpu.sync_copy(x_hbm.at[b], idx_v)
        return [pltpu.async_copy(table_hbm.at[idx_v.at[g]],
                                 rows_v.at[g], sem)
                for g in range(10)]

    def compute(rows_v, b):
        for si in range(NSUB):
            @pl.loop(0, SUB, unroll=4)
            def pos_loop(j, _si=si, _rows=rows_v):
                kr = [divmod(c * STEPS + _si * SUB, 80)
                      for c in range(NUM_CHANNEL)]
                acc = [ws[0] * _rows[kr[0][0], kr[0][1] + j, pl.ds(k * L, L)]
                       + ws[1] * _rows[kr[1][0], kr[1][1] + j, pl.ds(k * L, L)]
                       + ws[2] * _rows[kr[2][0], kr[2][1] + j, pl.ds(k * L, L)]
                       + ws[3] * _rows[kr[3][0], kr[3][1] + j, pl.ds(k * L, L)]
                       for k in range(HV)]
                tot = (acc[0] + acc[1]) + (acc[2] + acc[3])
                sq = (acc[0] * acc[0] + acc[1] * acc[1]) + \
                     (acc[2] * acc[2] + acc[3] * acc[3])
                mean = jnp.sum(tot) * (1.0 / HIDDEN)
                var = jnp.sum(sq) * (1.0 / HIDDEN) - mean * mean
                rstd = _rsqrt(jnp.full((L,), var + 1e-5, jnp.float32))
                s = _si * SUB + j
                for k in range(HV):
                    out_v[pl.ds(j * HIDDEN + k * L, L)] = (
                        (acc[k] - mean) * rstd * gam[k] + bet[k]
                        + pos_v[s, pl.ds(k * L, L)])

            pltpu.sync_copy(
                out_v, out_hbm.at[b, pl.ds(si * SUB * HIDDEN, SUB * HIDDEN)])

    b_base = wid * BATCH_PER_W

    @pl.loop(0, BATCH_PER_W // 2)
    def pair_loop(p):
        b0 = b_base + 2 * p
        cps0 = fire(b0, idx0, rows0, sem0)
        cps1 = fire(b0 + 1, idx1, rows1, sem1)
        for cp in cps0:
            cp.wait()
        compute(rows0, b0)
        for cp in cps1:
            cp.wait()
        compute(rows1, b0 + 1)


@jax.jit
def kernel(x, table, ch_w, ln_gamma, ln_beta, pos_emb):
    chw16 = jnp.full((L,), -1e30, jnp.float32).at[:NUM_CHANNEL].set(ch_w)

    mesh = plsc.VectorSubcoreMesh(core_axis_name="c", subcore_axis_name="s")
    run = pl.kernel(
        _body,
        out_type=jax.ShapeDtypeStruct((BATCH, STEPS * HIDDEN), jnp.float32),
        mesh=mesh,
        scratch_types=[
            pltpu.VMEM((10, 80), jnp.int32),
            pltpu.VMEM((10, 80), jnp.int32),
            pltpu.VMEM((10, 80, HIDDEN), jnp.float32),
            pltpu.VMEM((10, 80, HIDDEN), jnp.float32),
            pltpu.VMEM((SUB * HIDDEN,), jnp.float32),
            pltpu.VMEM((STEPS, HIDDEN), jnp.float32),
            pltpu.VMEM((2, HIDDEN), jnp.float32),
            pltpu.VMEM((L,), jnp.float32),
            pltpu.SemaphoreType.DMA,
            pltpu.SemaphoreType.DMA,
        ],
        compiler_params=pltpu.CompilerParams(
            needs_layout_passes=False, use_tc_tiling_on_sc=False),
    )
    x10 = x.reshape(BATCH, 10, 80)
    out = run(x10, chw16, ln_gamma, ln_beta, pos_emb, table)
    return out.reshape(BATCH, STEPS, HIDDEN)
